# Initial kernel scaffold; baseline (speedup 1.0000x reference)
import jax
import jax.numpy as jnp
from jax.experimental import pallas as pl


def kernel(inputs, token_table, position_table):
    raise NotImplementedError("write your pallas kernel here")


# speedup vs baseline: 4.1345x; 4.1345x over previous
import functools

import jax
import jax.numpy as jnp
from jax import lax
from jax.experimental import pallas as pl
from jax.experimental.pallas import tpu as pltpu
from jax.experimental.pallas import tpu_sc as plsc

B = 4096
L = 200
D = 64
LANES = 16
NC = 2
NS = 16
NW = NC * NS
SEQ_PER_W = B // NW
CH_SEQ = 4
ROWS = CH_SEQ * L
NCHUNK = SEQ_PER_W // CH_SEQ
NBUF = 2


def _emb_body(idx_hbm, pos_hbm, tok_hbm, out_hbm,
              pos_v, idx0, idx1, rows0, rows1, sem0, sem1):
    wid = lax.axis_index("s") * NC + lax.axis_index("c")
    w_base = wid * (SEQ_PER_W * L)

    pltpu.sync_copy(pos_hbm, pos_v)

    idx_b = (idx0, idx1)
    rows_b = (rows0, rows1)
    sem_b = (sem0, sem1)

    def fire(c, b):
        base = w_base + c * ROWS
        pltpu.sync_copy(idx_hbm.at[pl.ds(base, ROWS)], idx_b[b])
        pltpu.async_copy(tok_hbm.at[idx_b[b]], rows_b[b], sem_b[b])

    fire(0, 0)
    fire(1, 1)

    def outer(t, _):
        for b in range(NBUF):
            c = t * NBUF + b
            base = w_base + c * ROWS
            rows = rows_b[b]
            pltpu.make_async_copy(tok_hbm.at[idx_b[b]], rows, sem_b[b]).wait()

            def add_pos(i, _):
                for j in range(D // LANES):
                    p = pos_v[i, pl.ds(j * LANES, LANES)]
                    for k in range(CH_SEQ):
                        r = k * L + i
                        rows[r, pl.ds(j * LANES, LANES)] = (
                            rows[r, pl.ds(j * LANES, LANES)] + p)
                return 0

            lax.fori_loop(0, L, add_pos, 0, unroll=False)
            pltpu.sync_copy(rows, out_hbm.at[pl.ds(base, ROWS)])

            @pl.when(c + NBUF < NCHUNK)
            def _():
                fire(c + NBUF, b)
        return 0

    lax.fori_loop(0, NCHUNK // NBUF, outer, 0, unroll=False)


@jax.jit
def _emb(idx_flat, position_table, token_table):
    mesh = plsc.VectorSubcoreMesh(core_axis_name="c", subcore_axis_name="s")
    return pl.kernel(
        _emb_body,
        mesh=mesh,
        compiler_params=pltpu.CompilerParams(use_tc_tiling_on_sc=False),
        out_type=jax.ShapeDtypeStruct((B * L, D), jnp.float32),
        scratch_types=[
            pltpu.VMEM((L, D), jnp.float32),
            pltpu.VMEM((ROWS,), jnp.int32),
            pltpu.VMEM((ROWS,), jnp.int32),
            pltpu.VMEM((ROWS, D), jnp.float32),
            pltpu.VMEM((ROWS, D), jnp.float32),
            pltpu.SemaphoreType.DMA,
            pltpu.SemaphoreType.DMA,
        ],
    )(idx_flat, position_table, token_table)


def kernel(inputs, token_table, position_table):
    idx_flat = jnp.asarray(inputs, jnp.int32).reshape(B * L)
    out = _emb(idx_flat, position_table, token_table)
    return out.reshape(B, L, D)


# speedup vs baseline: 4.2445x; 1.0266x over previous
import functools

import jax
import jax.numpy as jnp
from jax import lax
from jax.experimental import pallas as pl
from jax.experimental.pallas import tpu as pltpu
from jax.experimental.pallas import tpu_sc as plsc

B = 4096
L = 200
D = 64
LANES = 16
NC = 2
NS = 16
NW = NC * NS
SEQ_PER_W = B // NW
CH_SEQ = 2
ROWS = CH_SEQ * L
NCHUNK = SEQ_PER_W // CH_SEQ
NBUF = 4


def _emb_body(idx_hbm, pos_hbm, tok_hbm, out_hbm, pos_v,
              idx0, idx1, idx2, idx3,
              rows0, rows1, rows2, rows3,
              gs0, gs1, gs2, gs3, ss0, ss1, ss2, ss3, is0, is1, is2, is3):
    wid = lax.axis_index("s") * NC + lax.axis_index("c")
    w_base = wid * (SEQ_PER_W * L)

    idx_b = (idx0, idx1, idx2, idx3)
    rows_b = (rows0, rows1, rows2, rows3)
    gsem = (gs0, gs1, gs2, gs3)
    ssem = (ss0, ss1, ss2, ss3)
    isem = (is0, is1, is2, is3)

    def fire_idx(c, b):
        pltpu.async_copy(
            idx_hbm.at[pl.ds(w_base + c * ROWS, ROWS)], idx_b[b], isem[b])

    def wait_idx(b):
        pltpu.make_async_copy(
            idx_hbm.at[pl.ds(0, ROWS)], idx_b[b], isem[b]).wait()

    def fire_gather(b):
        pltpu.async_copy(tok_hbm.at[idx_b[b]], rows_b[b], gsem[b])

    def wait_gather(b):
        pltpu.make_async_copy(tok_hbm.at[idx_b[b]], rows_b[b], gsem[b]).wait()

    def fire_store(c, b):
        pltpu.async_copy(
            rows_b[b], out_hbm.at[pl.ds(w_base + c * ROWS, ROWS)], ssem[b])

    def wait_store(b):
        pltpu.make_async_copy(
            rows_b[b], out_hbm.at[pl.ds(0, ROWS)], ssem[b]).wait()

    pltpu.sync_copy(pos_hbm, pos_v)

    fire_idx(0, 0)
    fire_idx(1, 1)
    fire_idx(2, 2)
    wait_idx(0)
    fire_gather(0)
    wait_idx(1)
    fire_gather(1)

    def outer(t, _):
        for b in range(NBUF):
            c = t * NBUF + b
            wait_gather(b)
            rows = rows_b[b]

            @plsc.parallel_loop(0, L, unroll=2)
            def _add(i):
                for j in range(D // LANES):
                    sl = pl.ds(j * LANES, LANES)
                    p = pos_v[i, sl]
                    for k in range(CH_SEQ):
                        rows[k * L + i, sl] = rows[k * L + i, sl] + p

            fire_store(c, b)

            bn = (b + 2) % NBUF

            @pl.when(c + 2 < NCHUNK)
            def _():
                @pl.when(c >= 2)
                def _():
                    wait_store(bn)
                wait_idx(bn)
                fire_gather(bn)

            @pl.when(c + 3 < NCHUNK)
            def _():
                fire_idx(c + 3, (b + 3) % NBUF)
        return 0

    lax.fori_loop(0, NCHUNK // NBUF, outer, 0, unroll=False)

    for b in range(NBUF):
        wait_store(b)


@jax.jit
def _emb(idx_flat, position_table, token_table):
    mesh = plsc.VectorSubcoreMesh(core_axis_name="c", subcore_axis_name="s")
    return pl.kernel(
        _emb_body,
        mesh=mesh,
        compiler_params=pltpu.CompilerParams(use_tc_tiling_on_sc=False),
        out_type=jax.ShapeDtypeStruct((B * L, D), jnp.float32),
        scratch_types=[
            pltpu.VMEM((L, D), jnp.float32),
            pltpu.VMEM((ROWS,), jnp.int32),
            pltpu.VMEM((ROWS,), jnp.int32),
            pltpu.VMEM((ROWS,), jnp.int32),
            pltpu.VMEM((ROWS,), jnp.int32),
            pltpu.VMEM((ROWS, D), jnp.float32),
            pltpu.VMEM((ROWS, D), jnp.float32),
            pltpu.VMEM((ROWS, D), jnp.float32),
            pltpu.VMEM((ROWS, D), jnp.float32),
        ] + [pltpu.SemaphoreType.DMA] * 12,
    )(idx_flat, position_table, token_table)


def kernel(inputs, token_table, position_table):
    idx_flat = jnp.asarray(inputs, jnp.int32).reshape(B * L)
    out = _emb(idx_flat, position_table, token_table)
    return out.reshape(B, L, D)
